# baseline (speedup 1.0000x reference)
import functools

import jax
import jax.numpy as jnp
from jax import lax
from jax.experimental import pallas as pl
from jax.experimental.pallas import tpu as pltpu
from jax.experimental.pallas import tpu_sc as plsc

N_CLS = 256
L = 16
NC, NS = 2, 16
NW = NC * NS

_LN2 = 0.6931471805599453
_SQRT2 = 1.4142135623730951


def _tree(op, xs):
    xs = list(xs)
    while len(xs) > 1:
        nxt = [op(xs[i], xs[i + 1]) for i in range(0, len(xs) - 1, 2)]
        if len(xs) % 2:
            nxt.append(xs[-1])
        xs = nxt
    return xs[0]


def _vlog(s):
    i = lax.bitcast_convert_type(s, jnp.int32)
    e = lax.shift_right_arithmetic(i, 23) - 127
    m = lax.bitcast_convert_type((i & 0x007FFFFF) | 0x3F800000, jnp.float32)
    big = m > _SQRT2
    m = jnp.where(big, 0.5 * m, m)
    ef = (e + jnp.where(big, 1, 0)).astype(jnp.float32)
    t = (m - 1.0) / (m + 1.0)
    t2 = t * t
    p = 2.0 + t2 * (2.0 / 3.0 + t2 * (2.0 / 5.0 + t2 * (2.0 / 7.0 + t2 * (2.0 / 9.0))))
    return ef * _LN2 + t * p


@functools.lru_cache(maxsize=None)
def _build(batch, ntok, nch):
    assert batch == NW
    D = nch * N_CLS
    NCT = D // 128
    TCH = 32 if ntok % 64 == 0 else ntok // 2
    NCHK = ntok // TCH
    assert NCHK % 2 == 0 and TCH % L == 0

    mesh = plsc.VectorSubcoreMesh(
        core_axis_name="c", subcore_axis_name="s",
        num_cores=NC, num_subcores=NS)

    @functools.partial(
        pl.kernel,
        out_type=jax.ShapeDtypeStruct((batch, ntok, nch), jnp.float32),
        mesh=mesh,
        compiler_params=pltpu.CompilerParams(needs_layout_passes=False),
        scratch_types=[
            [pltpu.VMEM((NCT, TCH, 128), jnp.float32) for _ in range(2)],
            [pltpu.VMEM((TCH, nch), jnp.int32) for _ in range(2)],
            [pltpu.VMEM((TCH, nch), jnp.float32) for _ in range(2)],
            [pltpu.SemaphoreType.DMA for _ in range(2)],
            [pltpu.SemaphoreType.DMA for _ in range(2)],
        ],
    )
    def sc_kernel(lp_hbm, x_hbm, out_hbm, bufs, xbufs, obufs, isems, osems):
        wid = lax.axis_index("s") * NC + lax.axis_index("c")
        lane = lax.iota(jnp.int32, L)

        def in_start(ci, p):
            for ct in range(NCT):
                pltpu.async_copy(
                    lp_hbm.at[wid, pl.ds(ci * TCH, TCH), pl.ds(128 * ct, 128)],
                    bufs[p].at[ct], isems[p])
            pltpu.async_copy(x_hbm.at[wid, pl.ds(ci * TCH, TCH)],
                             xbufs[p], isems[p])

        def in_wait(p):
            for ct in range(NCT):
                pltpu.make_async_copy(
                    lp_hbm.at[0, pl.ds(0, TCH), pl.ds(128 * ct, 128)],
                    bufs[p].at[ct], isems[p]).wait()
            pltpu.make_async_copy(x_hbm.at[0, pl.ds(0, TCH)],
                                  xbufs[p], isems[p]).wait()

        def out_start(ci, p):
            pltpu.async_copy(obufs[p],
                             out_hbm.at[wid, pl.ds(ci * TCH, TCH)], osems[p])

        def out_wait(p):
            pltpu.make_async_copy(obufs[p],
                                  out_hbm.at[0, pl.ds(0, TCH)],
                                  osems[p]).wait()

        def process(p):
            buf, xbuf, obuf = bufs[p], xbufs[p], obufs[p]

            @plsc.parallel_loop(0, (TCH // L) * nch, unroll=2)
            def gbody(g):
                ch = g % nch
                tok0 = (g // nch) * L
                toks = tok0 + lane
                ct0 = 2 * ch
                acc_m = jnp.zeros((L,), jnp.float32)
                acc_s = jnp.zeros((L,), jnp.float32)
                for t in range(L):
                    vals = [buf[ct0 + j // 8, tok0 + t, pl.ds((j % 8) * L, L)]
                            for j in range(N_CLS // L)]
                    mx = jnp.max(_tree(jnp.maximum, vals))
                    ssum = jnp.sum(_tree(jnp.add,
                                         [jnp.exp(v - mx) for v in vals]))
                    sel = lane == t
                    acc_m = jnp.where(sel, mx, acc_m)
                    acc_s = jnp.where(sel, ssum, acc_s)
                lse = acc_m + _vlog(acc_s)
                chv = jnp.full((L,), 0, jnp.int32) + ch
                xi = plsc.load_gather(xbuf, [toks, chv])
                ctv = ct0 + lax.shift_right_logical(xi, 7)
                picked = plsc.load_gather(buf, [ctv, toks, xi & 127])
                plsc.store_scatter(obuf, [toks, chv], picked - lse)

        in_start(0, 0)
        in_start(1, 1)

        def cbody(k, carry):
            i = 2 * k
            for p in range(2):
                @pl.when(i >= 2)
                def _():
                    out_wait(p)

                in_wait(p)
                process(p)
                out_start(i + p, p)

                @pl.when(i + 2 + p < NCHK)
                def _():
                    in_start(i + 2 + p, p)
            return carry

        lax.fori_loop(0, NCHK // 2, cbody, 0)
        out_wait(0)
        out_wait(1)

    return sc_kernel


def kernel(x, dist_params):
    b, t, ch = x.shape
    return _build(b, t, ch)(dist_params, x)


# speedup vs baseline: 1.3386x; 1.3386x over previous
import functools

import jax
import jax.numpy as jnp
from jax import lax
from jax.experimental import pallas as pl
from jax.experimental.pallas import tpu as pltpu
from jax.experimental.pallas import tpu_sc as plsc

N_CLS = 256
L = 16
NC, NS = 2, 16
NW = NC * NS

_LN2 = 0.6931471805599453
_SQRT2 = 1.4142135623730951


def _tree(op, xs):
    xs = list(xs)
    while len(xs) > 1:
        nxt = [op(xs[i], xs[i + 1]) for i in range(0, len(xs) - 1, 2)]
        if len(xs) % 2:
            nxt.append(xs[-1])
        xs = nxt
    return xs[0]


def _vlog(s):
    i = lax.bitcast_convert_type(s, jnp.int32)
    e = lax.shift_right_arithmetic(i, 23) - 127
    m = lax.bitcast_convert_type((i & 0x007FFFFF) | 0x3F800000, jnp.float32)
    big = m > _SQRT2
    m = jnp.where(big, 0.5 * m, m)
    ef = (e + jnp.where(big, 1, 0)).astype(jnp.float32)
    t = (m - 1.0) / (m + 1.0)
    t2 = t * t
    p = 2.0 + t2 * (2.0 / 3.0 + t2 * (2.0 / 5.0 + t2 * (2.0 / 7.0 + t2 * (2.0 / 9.0))))
    return ef * _LN2 + t * p


@functools.lru_cache(maxsize=None)
def _build(batch, ntok, nch):
    assert batch == NW
    D = nch * N_CLS
    NCT = D // 128
    TCH = 32 if ntok % 64 == 0 else ntok // 2
    NCHK = ntok // TCH
    assert NCHK % 2 == 0 and TCH % L == 0

    mesh = plsc.VectorSubcoreMesh(
        core_axis_name="c", subcore_axis_name="s",
        num_cores=NC, num_subcores=NS)

    @functools.partial(
        pl.kernel,
        out_type=jax.ShapeDtypeStruct((batch, ntok, nch), jnp.float32),
        mesh=mesh,
        compiler_params=pltpu.CompilerParams(needs_layout_passes=False),
        scratch_types=[
            [pltpu.VMEM((NCT, TCH, 128), jnp.float32) for _ in range(2)],
            [pltpu.VMEM((TCH, nch), jnp.int32) for _ in range(2)],
            [pltpu.VMEM((TCH, nch), jnp.float32) for _ in range(2)],
            [pltpu.SemaphoreType.DMA for _ in range(2)],
            [pltpu.SemaphoreType.DMA for _ in range(2)],
        ],
    )
    def sc_kernel(lp_hbm, x_hbm, out_hbm, bufs, xbufs, obufs, isems, osems):
        wid = lax.axis_index("s") * NC + lax.axis_index("c")
        lane = lax.iota(jnp.int32, L)

        def in_start(ci, p):
            for ct in range(NCT):
                pltpu.async_copy(
                    lp_hbm.at[wid, pl.ds(ci * TCH, TCH), pl.ds(128 * ct, 128)],
                    bufs[p].at[ct], isems[p])
            pltpu.async_copy(x_hbm.at[wid, pl.ds(ci * TCH, TCH)],
                             xbufs[p], isems[p])

        def in_wait(p):
            for ct in range(NCT):
                pltpu.make_async_copy(
                    lp_hbm.at[0, pl.ds(0, TCH), pl.ds(128 * ct, 128)],
                    bufs[p].at[ct], isems[p]).wait()
            pltpu.make_async_copy(x_hbm.at[0, pl.ds(0, TCH)],
                                  xbufs[p], isems[p]).wait()

        def out_start(ci, p):
            pltpu.async_copy(obufs[p],
                             out_hbm.at[wid, pl.ds(ci * TCH, TCH)], osems[p])

        def out_wait(p):
            pltpu.make_async_copy(obufs[p],
                                  out_hbm.at[0, pl.ds(0, TCH)],
                                  osems[p]).wait()

        def process(p):
            buf, xbuf, obuf = bufs[p], xbufs[p], obufs[p]

            @plsc.parallel_loop(0, (TCH // L) * nch, unroll=1)
            def gbody(g):
                ch = g % nch
                tok0 = (g // nch) * L
                toks = tok0 + lane
                ct0 = 2 * ch
                acc_m = jnp.zeros((L,), jnp.float32)
                acc_s = jnp.zeros((L,), jnp.float32)
                for t in range(L):
                    vals = [buf[ct0 + j // 8, tok0 + t, pl.ds((j % 8) * L, L)]
                            for j in range(N_CLS // L)]
                    mx = jnp.max(_tree(jnp.maximum, vals))
                    ssum = jnp.sum(_tree(jnp.add,
                                         [jnp.exp(v - mx) for v in vals]))
                    sel = lane == t
                    acc_m = jnp.where(sel, mx, acc_m)
                    acc_s = jnp.where(sel, ssum, acc_s)
                lse = acc_m + _vlog(acc_s)
                chv = jnp.full((L,), 0, jnp.int32) + ch
                xi = plsc.load_gather(xbuf, [toks, chv])
                ctv = ct0 + lax.shift_right_logical(xi, 7)
                picked = plsc.load_gather(buf, [ctv, toks, xi & 127])
                plsc.store_scatter(obuf, [toks, chv], picked - lse)

        in_start(0, 0)
        in_start(1, 1)

        def cbody(k, carry):
            i = 2 * k
            for p in range(2):
                @pl.when(i >= 2)
                def _():
                    out_wait(p)

                in_wait(p)
                process(p)
                out_start(i + p, p)

                @pl.when(i + 2 + p < NCHK)
                def _():
                    in_start(i + 2 + p, p)
            return carry

        lax.fori_loop(0, NCHK // 2, cbody, 0)
        out_wait(0)
        out_wait(1)

    return sc_kernel


def kernel(x, dist_params):
    b, t, ch = x.shape
    return _build(b, t, ch)(dist_params, x)


# speedup vs baseline: 1.5883x; 1.1866x over previous
import functools

import jax
import jax.numpy as jnp
from jax import lax
from jax.experimental import pallas as pl
from jax.experimental.pallas import tpu as pltpu
from jax.experimental.pallas import tpu_sc as plsc

N_CLS = 256
L = 16
NC, NS = 2, 16
NW = NC * NS

_LN2 = 0.6931471805599453
_SQRT2 = 1.4142135623730951


def _tree(op, xs):
    xs = list(xs)
    while len(xs) > 1:
        nxt = [op(xs[i], xs[i + 1]) for i in range(0, len(xs) - 1, 2)]
        if len(xs) % 2:
            nxt.append(xs[-1])
        xs = nxt
    return xs[0]


def _vlog(s):
    i = lax.bitcast_convert_type(s, jnp.int32)
    e = lax.shift_right_arithmetic(i, 23) - 127
    m = lax.bitcast_convert_type((i & 0x007FFFFF) | 0x3F800000, jnp.float32)
    big = m > _SQRT2
    m = jnp.where(big, 0.5 * m, m)
    ef = (e + jnp.where(big, 1, 0)).astype(jnp.float32)
    t = (m - 1.0) / (m + 1.0)
    t2 = t * t
    p = 2.0 + t2 * (2.0 / 3.0 + t2 * (2.0 / 5.0 + t2 * (2.0 / 7.0 + t2 * (2.0 / 9.0))))
    return ef * _LN2 + t * p


@functools.lru_cache(maxsize=None)
def _build(tsc, nch):
    D = nch * N_CLS
    NCT = D // 128
    ntok = tsc // NW
    TCH = 32 if ntok % 64 == 0 else ntok // 2
    NCHK = ntok // TCH
    assert NCHK % 2 == 0 and TCH % L == 0

    mesh = plsc.VectorSubcoreMesh(
        core_axis_name="c", subcore_axis_name="s",
        num_cores=NC, num_subcores=NS)

    @functools.partial(
        pl.kernel,
        out_type=jax.ShapeDtypeStruct((tsc, nch), jnp.float32),
        mesh=mesh,
        compiler_params=pltpu.CompilerParams(needs_layout_passes=False),
        scratch_types=[
            [pltpu.VMEM((NCT, TCH, 128), jnp.float32) for _ in range(2)],
            [pltpu.VMEM((TCH, nch), jnp.int32) for _ in range(2)],
            [pltpu.VMEM((TCH, nch), jnp.float32) for _ in range(2)],
            [pltpu.SemaphoreType.DMA for _ in range(2)],
            [pltpu.SemaphoreType.DMA for _ in range(2)],
        ],
    )
    def sc_kernel(lp_hbm, x_hbm, out_hbm, bufs, xbufs, obufs, isems, osems):
        wid = lax.axis_index("s") * NC + lax.axis_index("c")
        base = wid * ntok
        lane = lax.iota(jnp.int32, L)

        def in_start(ci, p):
            for ct in range(NCT):
                pltpu.async_copy(
                    lp_hbm.at[pl.ds(base + ci * TCH, TCH),
                              pl.ds(128 * ct, 128)],
                    bufs[p].at[ct], isems[p])
            pltpu.async_copy(x_hbm.at[pl.ds(base + ci * TCH, TCH)],
                             xbufs[p], isems[p])

        def in_wait(p):
            for ct in range(NCT):
                pltpu.make_async_copy(
                    lp_hbm.at[pl.ds(0, TCH), pl.ds(128 * ct, 128)],
                    bufs[p].at[ct], isems[p]).wait()
            pltpu.make_async_copy(x_hbm.at[pl.ds(0, TCH)],
                                  xbufs[p], isems[p]).wait()

        def out_start(ci, p):
            pltpu.async_copy(obufs[p],
                             out_hbm.at[pl.ds(base + ci * TCH, TCH)], osems[p])

        def out_wait(p):
            pltpu.make_async_copy(obufs[p],
                                  out_hbm.at[pl.ds(0, TCH)],
                                  osems[p]).wait()

        def process(p):
            buf, xbuf, obuf = bufs[p], xbufs[p], obufs[p]

            @plsc.parallel_loop(0, (TCH // L) * nch, unroll=1)
            def gbody(g):
                ch = g % nch
                tok0 = (g // nch) * L
                toks = tok0 + lane
                ct0 = 2 * ch
                acc_m = jnp.zeros((L,), jnp.float32)
                acc_s = jnp.zeros((L,), jnp.float32)
                for t in range(L):
                    vals = [buf[ct0 + j // 8, tok0 + t, pl.ds((j % 8) * L, L)]
                            for j in range(N_CLS // L)]
                    mx = jnp.max(_tree(jnp.maximum, vals))
                    ssum = jnp.sum(_tree(jnp.add,
                                         [jnp.exp(v - mx) for v in vals]))
                    sel = lane == t
                    acc_m = jnp.where(sel, mx, acc_m)
                    acc_s = jnp.where(sel, ssum, acc_s)
                lse = acc_m + _vlog(acc_s)
                chv = jnp.full((L,), 0, jnp.int32) + ch
                xi = plsc.load_gather(xbuf, [toks, chv])
                ctv = ct0 + lax.shift_right_logical(xi, 7)
                picked = plsc.load_gather(buf, [ctv, toks, xi & 127])
                plsc.store_scatter(obuf, [toks, chv], picked - lse)

        in_start(0, 0)
        in_start(1, 1)

        def cbody(k, carry):
            i = 2 * k
            for p in range(2):
                @pl.when(i >= 2)
                def _():
                    out_wait(p)

                in_wait(p)
                process(p)
                out_start(i + p, p)

                @pl.when(i + 2 + p < NCHK)
                def _():
                    in_start(i + 2 + p, p)
            return carry

        lax.fori_loop(0, NCHK // 2, cbody, 0)
        out_wait(0)
        out_wait(1)

    return sc_kernel


_BT = 512


@functools.lru_cache(maxsize=None)
def _build_tc(ttc, off_blocks, nch):
    D = nch * N_CLS
    assert ttc % _BT == 0

    def body(x_ref, lp_ref, o_ref):
        blk = lp_ref[...]
        xb = x_ref[...]
        outs = []
        for ch in range(nch):
            sub = blk[:, ch * N_CLS:(ch + 1) * N_CLS]
            m = jnp.max(sub, axis=1, keepdims=True)
            e = jnp.exp(sub - m)
            lse = m + jnp.log(jnp.sum(e, axis=1, keepdims=True))
            idx = lax.broadcasted_iota(jnp.int32, (_BT, N_CLS), 1)
            sel = idx == xb[:, ch][:, None]
            picked = jnp.sum(jnp.where(sel, sub, 0.0), axis=1, keepdims=True)
            outs.append(picked - lse)
        o_ref[...] = jnp.concatenate(outs, axis=1)

    return pl.pallas_call(
        body,
        out_shape=jax.ShapeDtypeStruct((ttc, nch), jnp.float32),
        grid=(ttc // _BT,),
        in_specs=[
            pl.BlockSpec((_BT, nch), lambda i: (off_blocks + i, 0)),
            pl.BlockSpec((_BT, D), lambda i: (off_blocks + i, 0)),
        ],
        out_specs=pl.BlockSpec((_BT, nch), lambda i: (i, 0)),
    )


def kernel(x, dist_params):
    b, t, ch = x.shape
    tt = b * t
    lp = dist_params.reshape(tt, ch * N_CLS)
    xf = x.reshape(tt, ch)
    tsc = (tt // 2) // (NW * 64) * (NW * 64)
    sc_out = _build(tsc, ch)(lp, xf)
    tc_out = _build_tc(tt - tsc, tsc // _BT, ch)(xf, lp)
    return jnp.concatenate([sc_out, tc_out], axis=0).reshape(b, t, ch)


# speedup vs baseline: 1.7621x; 1.1094x over previous
import functools

import jax
import jax.numpy as jnp
from jax import lax
from jax.experimental import pallas as pl
from jax.experimental.pallas import tpu as pltpu
from jax.experimental.pallas import tpu_sc as plsc

N_CLS = 256
L = 16
NC, NS = 2, 16
NW = NC * NS

_LN2 = 0.6931471805599453
_SQRT2 = 1.4142135623730951


def _tree(op, xs):
    xs = list(xs)
    while len(xs) > 1:
        nxt = [op(xs[i], xs[i + 1]) for i in range(0, len(xs) - 1, 2)]
        if len(xs) % 2:
            nxt.append(xs[-1])
        xs = nxt
    return xs[0]


def _vlog(s):
    i = lax.bitcast_convert_type(s, jnp.int32)
    e = lax.shift_right_arithmetic(i, 23) - 127
    m = lax.bitcast_convert_type((i & 0x007FFFFF) | 0x3F800000, jnp.float32)
    big = m > _SQRT2
    m = jnp.where(big, 0.5 * m, m)
    ef = (e + jnp.where(big, 1, 0)).astype(jnp.float32)
    t = (m - 1.0) / (m + 1.0)
    t2 = t * t
    p = 2.0 + t2 * (2.0 / 3.0 + t2 * (2.0 / 5.0 + t2 * (2.0 / 7.0 + t2 * (2.0 / 9.0))))
    return ef * _LN2 + t * p


@functools.lru_cache(maxsize=None)
def _build(tsc, nch):
    D = nch * N_CLS
    NCT = D // 128
    ntok = tsc // NW
    TCH = 32 if ntok % 64 == 0 else ntok // 2
    NCHK = ntok // TCH
    assert NCHK % 2 == 0 and TCH % L == 0

    mesh = plsc.VectorSubcoreMesh(
        core_axis_name="c", subcore_axis_name="s",
        num_cores=NC, num_subcores=NS)

    @functools.partial(
        pl.kernel,
        out_type=jax.ShapeDtypeStruct((tsc, nch), jnp.float32),
        mesh=mesh,
        compiler_params=pltpu.CompilerParams(needs_layout_passes=False),
        scratch_types=[
            [pltpu.VMEM((NCT, TCH, 128), jnp.float32) for _ in range(2)],
            [pltpu.VMEM((TCH, nch), jnp.int32) for _ in range(2)],
            [pltpu.VMEM((TCH, nch), jnp.float32) for _ in range(2)],
            [pltpu.SemaphoreType.DMA for _ in range(2)],
            [pltpu.SemaphoreType.DMA for _ in range(2)],
        ],
    )
    def sc_kernel(lp_hbm, x_hbm, out_hbm, bufs, xbufs, obufs, isems, osems):
        wid = lax.axis_index("s") * NC + lax.axis_index("c")
        base = wid * ntok
        lane = lax.iota(jnp.int32, L)

        def in_start(ci, p):
            for ct in range(NCT):
                pltpu.async_copy(
                    lp_hbm.at[pl.ds(base + ci * TCH, TCH),
                              pl.ds(128 * ct, 128)],
                    bufs[p].at[ct], isems[p])
            pltpu.async_copy(x_hbm.at[pl.ds(base + ci * TCH, TCH)],
                             xbufs[p], isems[p])

        def in_wait(p):
            for ct in range(NCT):
                pltpu.make_async_copy(
                    lp_hbm.at[pl.ds(0, TCH), pl.ds(128 * ct, 128)],
                    bufs[p].at[ct], isems[p]).wait()
            pltpu.make_async_copy(x_hbm.at[pl.ds(0, TCH)],
                                  xbufs[p], isems[p]).wait()

        def out_start(ci, p):
            pltpu.async_copy(obufs[p],
                             out_hbm.at[pl.ds(base + ci * TCH, TCH)], osems[p])

        def out_wait(p):
            pltpu.make_async_copy(obufs[p],
                                  out_hbm.at[pl.ds(0, TCH)],
                                  osems[p]).wait()

        def process(p):
            buf, xbuf, obuf = bufs[p], xbufs[p], obufs[p]

            @plsc.parallel_loop(0, (TCH // L) * nch, unroll=1)
            def gbody(g):
                ch = g % nch
                tok0 = (g // nch) * L
                toks = tok0 + lane
                ct0 = 2 * ch
                acc_m = jnp.zeros((L,), jnp.float32)
                acc_s = jnp.zeros((L,), jnp.float32)
                for t in range(L):
                    vals = [buf[ct0 + j // 8, tok0 + t, pl.ds((j % 8) * L, L)]
                            for j in range(N_CLS // L)]
                    mx = jnp.max(_tree(jnp.maximum, vals))
                    ssum = jnp.sum(_tree(jnp.add,
                                         [jnp.exp(v - mx) for v in vals]))
                    sel = lane == t
                    acc_m = jnp.where(sel, mx, acc_m)
                    acc_s = jnp.where(sel, ssum, acc_s)
                lse = acc_m + _vlog(acc_s)
                chv = jnp.full((L,), 0, jnp.int32) + ch
                xi = plsc.load_gather(xbuf, [toks, chv])
                ctv = ct0 + lax.shift_right_logical(xi, 7)
                picked = plsc.load_gather(buf, [ctv, toks, xi & 127])
                plsc.store_scatter(obuf, [toks, chv], picked - lse)

        in_start(0, 0)
        in_start(1, 1)

        def cbody(k, carry):
            i = 2 * k
            for p in range(2):
                @pl.when(i >= 2)
                def _():
                    out_wait(p)

                in_wait(p)
                process(p)
                out_start(i + p, p)

                @pl.when(i + 2 + p < NCHK)
                def _():
                    in_start(i + 2 + p, p)
            return carry

        lax.fori_loop(0, NCHK // 2, cbody, 0)
        out_wait(0)
        out_wait(1)

    return sc_kernel


_BT = 512


@functools.lru_cache(maxsize=None)
def _build_tc(ttc, off_blocks, nch):
    D = nch * N_CLS
    assert ttc % _BT == 0

    def body(x_ref, lp_ref, o_ref):
        blk = lp_ref[...]
        xb = x_ref[...]
        idx = lax.broadcasted_iota(jnp.int32, (_BT, N_CLS), 1)
        outs = []
        for ch in range(nch):
            sub = blk[:, ch * N_CLS:(ch + 1) * N_CLS]
            m = jnp.max(sub, axis=1, keepdims=True)
            e = jnp.exp(sub - m)
            lse = m + jnp.log(jnp.sum(e, axis=1, keepdims=True))
            sel = idx == xb[:, ch][:, None]
            picked = jnp.sum(jnp.where(sel, sub, 0.0), axis=1, keepdims=True)
            outs.append(picked - lse)
        o_ref[...] = jnp.concatenate(outs, axis=1)

    return pl.pallas_call(
        body,
        out_shape=jax.ShapeDtypeStruct((ttc, nch), jnp.float32),
        grid=(ttc // _BT,),
        in_specs=[
            pl.BlockSpec((_BT, nch), lambda i: (off_blocks + i, 0)),
            pl.BlockSpec((_BT, D), lambda i: (off_blocks + i, 0)),
        ],
        out_specs=pl.BlockSpec((_BT, nch), lambda i: (i, 0)),
    )


def kernel(x, dist_params):
    b, t, ch = x.shape
    tt = b * t
    lp = dist_params.reshape(tt, ch * N_CLS)
    xf = x.reshape(tt, ch)
    tsc = (tt * 19 // 32) // (NW * 64) * (NW * 64)
    sc_out = _build(tsc, ch)(lp, xf)
    tc_out = _build_tc(tt - tsc, tsc // _BT, ch)(xf, lp)
    return jnp.concatenate([sc_out, tc_out], axis=0).reshape(b, t, ch)


# speedup vs baseline: 1.8287x; 1.0378x over previous
import functools

import jax
import jax.numpy as jnp
from jax import lax
from jax.experimental import pallas as pl
from jax.experimental.pallas import tpu as pltpu
from jax.experimental.pallas import tpu_sc as plsc

N_CLS = 256
L = 16
NC, NS = 2, 16
NW = NC * NS

_LN2 = 0.6931471805599453
_SQRT2 = 1.4142135623730951


def _tree(op, xs):
    xs = list(xs)
    while len(xs) > 1:
        nxt = [op(xs[i], xs[i + 1]) for i in range(0, len(xs) - 1, 2)]
        if len(xs) % 2:
            nxt.append(xs[-1])
        xs = nxt
    return xs[0]


def _vlog(s):
    i = lax.bitcast_convert_type(s, jnp.int32)
    e = lax.shift_right_arithmetic(i, 23) - 127
    m = lax.bitcast_convert_type((i & 0x007FFFFF) | 0x3F800000, jnp.float32)
    big = m > _SQRT2
    m = jnp.where(big, 0.5 * m, m)
    ef = (e + jnp.where(big, 1, 0)).astype(jnp.float32)
    t = (m - 1.0) / (m + 1.0)
    t2 = t * t
    p = 2.0 + t2 * (2.0 / 3.0 + t2 * (2.0 / 5.0 + t2 * (2.0 / 7.0 + t2 * (2.0 / 9.0))))
    return ef * _LN2 + t * p


@functools.lru_cache(maxsize=None)
def _build(tsc, nch):
    D = nch * N_CLS
    NCT = D // 128
    ntok = tsc // NW
    TCH = 32 if ntok % 64 == 0 else ntok // 2
    NCHK = ntok // TCH
    assert NCHK % 2 == 0 and TCH % L == 0

    mesh = plsc.VectorSubcoreMesh(
        core_axis_name="c", subcore_axis_name="s",
        num_cores=NC, num_subcores=NS)

    @functools.partial(
        pl.kernel,
        out_type=jax.ShapeDtypeStruct((tsc, nch), jnp.float32),
        mesh=mesh,
        compiler_params=pltpu.CompilerParams(needs_layout_passes=False),
        scratch_types=[
            [pltpu.VMEM((NCT, TCH, 128), jnp.float32) for _ in range(2)],
            [pltpu.VMEM((TCH, nch), jnp.int32) for _ in range(2)],
            [pltpu.VMEM((TCH, nch), jnp.float32) for _ in range(2)],
            [pltpu.SemaphoreType.DMA for _ in range(2)],
            [pltpu.SemaphoreType.DMA for _ in range(2)],
        ],
    )
    def sc_kernel(lp_hbm, x_hbm, out_hbm, bufs, xbufs, obufs, isems, osems):
        wid = lax.axis_index("s") * NC + lax.axis_index("c")
        base = wid * ntok
        lane = lax.iota(jnp.int32, L)

        def in_start(ci, p):
            for ct in range(NCT):
                pltpu.async_copy(
                    lp_hbm.at[pl.ds(base + ci * TCH, TCH),
                              pl.ds(128 * ct, 128)],
                    bufs[p].at[ct], isems[p])
            pltpu.async_copy(x_hbm.at[pl.ds(base + ci * TCH, TCH)],
                             xbufs[p], isems[p])

        def in_wait(p):
            for ct in range(NCT):
                pltpu.make_async_copy(
                    lp_hbm.at[pl.ds(0, TCH), pl.ds(128 * ct, 128)],
                    bufs[p].at[ct], isems[p]).wait()
            pltpu.make_async_copy(x_hbm.at[pl.ds(0, TCH)],
                                  xbufs[p], isems[p]).wait()

        def out_start(ci, p):
            pltpu.async_copy(obufs[p],
                             out_hbm.at[pl.ds(base + ci * TCH, TCH)], osems[p])

        def out_wait(p):
            pltpu.make_async_copy(obufs[p],
                                  out_hbm.at[pl.ds(0, TCH)],
                                  osems[p]).wait()

        def process(p):
            buf, xbuf, obuf = bufs[p], xbufs[p], obufs[p]

            @plsc.parallel_loop(0, (TCH // L) * nch, unroll=1)
            def gbody(g):
                ch = g % nch
                tok0 = (g // nch) * L
                toks = tok0 + lane
                ct0 = 2 * ch
                acc_m = jnp.zeros((L,), jnp.float32)
                acc_s = jnp.zeros((L,), jnp.float32)
                for t in range(L):
                    vals = [buf[ct0 + j // 8, tok0 + t, pl.ds((j % 8) * L, L)]
                            for j in range(N_CLS // L)]
                    mx = jnp.max(_tree(jnp.maximum, vals))
                    ssum = jnp.sum(_tree(jnp.add,
                                         [jnp.exp(v - mx) for v in vals]))
                    sel = lane == t
                    acc_m = jnp.where(sel, mx, acc_m)
                    acc_s = jnp.where(sel, ssum, acc_s)
                lse = acc_m + _vlog(acc_s)
                chv = jnp.full((L,), 0, jnp.int32) + ch
                xi = plsc.load_gather(xbuf, [toks, chv])
                ctv = ct0 + lax.shift_right_logical(xi, 7)
                picked = plsc.load_gather(buf, [ctv, toks, xi & 127])
                plsc.store_scatter(obuf, [toks, chv], picked - lse)

        in_start(0, 0)
        in_start(1, 1)

        def cbody(k, carry):
            i = 2 * k
            for p in range(2):
                @pl.when(i >= 2)
                def _():
                    out_wait(p)

                in_wait(p)
                process(p)
                out_start(i + p, p)

                @pl.when(i + 2 + p < NCHK)
                def _():
                    in_start(i + 2 + p, p)
            return carry

        lax.fori_loop(0, NCHK // 2, cbody, 0)
        out_wait(0)
        out_wait(1)

    return sc_kernel


_BT = 1024


@functools.lru_cache(maxsize=None)
def _build_tc(ttc, off_blocks, nch):
    D = nch * N_CLS
    assert ttc % _BT == 0

    def body(x_ref, lp_ref, o_ref):
        blk = lp_ref[...]
        xb = x_ref[...]
        idx = lax.broadcasted_iota(jnp.int32, (_BT, N_CLS), 1)
        outs = []
        for ch in range(nch):
            sub = blk[:, ch * N_CLS:(ch + 1) * N_CLS]
            m = jnp.max(sub, axis=1, keepdims=True)
            e = jnp.exp(sub - m)
            lse = m + jnp.log(jnp.sum(e, axis=1, keepdims=True))
            sel = idx == xb[:, ch][:, None]
            picked = jnp.sum(jnp.where(sel, sub, 0.0), axis=1, keepdims=True)
            outs.append(picked - lse)
        o_ref[...] = jnp.concatenate(outs, axis=1)

    return pl.pallas_call(
        body,
        out_shape=jax.ShapeDtypeStruct((ttc, nch), jnp.float32),
        grid=(ttc // _BT,),
        in_specs=[
            pl.BlockSpec((_BT, nch), lambda i: (off_blocks + i, 0)),
            pl.BlockSpec((_BT, D), lambda i: (off_blocks + i, 0)),
        ],
        out_specs=pl.BlockSpec((_BT, nch), lambda i: (i, 0)),
    )


def kernel(x, dist_params):
    b, t, ch = x.shape
    tt = b * t
    lp = dist_params.reshape(tt, ch * N_CLS)
    xf = x.reshape(tt, ch)
    tsc = (tt * 19 // 32) // (NW * 64) * (NW * 64)
    sc_out = _build(tsc, ch)(lp, xf)
    tc_out = _build_tc(tt - tsc, tsc // _BT, ch)(xf, lp)
    return jnp.concatenate([sc_out, tc_out], axis=0).reshape(b, t, ch)


# speedup vs baseline: 1.8395x; 1.0059x over previous
import functools

import jax
import jax.numpy as jnp
from jax import lax
from jax.experimental import pallas as pl
from jax.experimental.pallas import tpu as pltpu
from jax.experimental.pallas import tpu_sc as plsc

N_CLS = 256
L = 16
NC, NS = 2, 16
NW = NC * NS

_LN2 = 0.6931471805599453
_SQRT2 = 1.4142135623730951


def _tree(op, xs):
    xs = list(xs)
    while len(xs) > 1:
        nxt = [op(xs[i], xs[i + 1]) for i in range(0, len(xs) - 1, 2)]
        if len(xs) % 2:
            nxt.append(xs[-1])
        xs = nxt
    return xs[0]


def _vlog(s):
    i = lax.bitcast_convert_type(s, jnp.int32)
    e = lax.shift_right_arithmetic(i, 23) - 127
    m = lax.bitcast_convert_type((i & 0x007FFFFF) | 0x3F800000, jnp.float32)
    big = m > _SQRT2
    m = jnp.where(big, 0.5 * m, m)
    ef = (e + jnp.where(big, 1, 0)).astype(jnp.float32)
    t = (m - 1.0) / (m + 1.0)
    t2 = t * t
    p = 2.0 + t2 * (2.0 / 3.0 + t2 * (2.0 / 5.0 + t2 * (2.0 / 7.0 + t2 * (2.0 / 9.0))))
    return ef * _LN2 + t * p


@functools.lru_cache(maxsize=None)
def _build(tsc, nch):
    D = nch * N_CLS
    NCT = D // 128
    ntok = tsc // NW
    TCH = 32 if ntok % 64 == 0 else ntok // 2
    NCHK = ntok // TCH
    assert NCHK % 2 == 0 and TCH % L == 0

    mesh = plsc.VectorSubcoreMesh(
        core_axis_name="c", subcore_axis_name="s",
        num_cores=NC, num_subcores=NS)

    @functools.partial(
        pl.kernel,
        out_type=jax.ShapeDtypeStruct((tsc, nch), jnp.float32),
        mesh=mesh,
        compiler_params=pltpu.CompilerParams(needs_layout_passes=False),
        scratch_types=[
            [pltpu.VMEM((NCT, TCH, 128), jnp.float32) for _ in range(2)],
            [pltpu.VMEM((TCH, nch), jnp.int32) for _ in range(2)],
            [pltpu.VMEM((TCH, nch), jnp.float32) for _ in range(2)],
            [pltpu.SemaphoreType.DMA for _ in range(2)],
            [pltpu.SemaphoreType.DMA for _ in range(2)],
        ],
    )
    def sc_kernel(lp_hbm, x_hbm, out_hbm, bufs, xbufs, obufs, isems, osems):
        wid = lax.axis_index("s") * NC + lax.axis_index("c")
        base = wid * ntok
        lane = lax.iota(jnp.int32, L)

        def in_start(ci, p):
            for ct in range(NCT):
                pltpu.async_copy(
                    lp_hbm.at[pl.ds(base + ci * TCH, TCH),
                              pl.ds(128 * ct, 128)],
                    bufs[p].at[ct], isems[p])
            pltpu.async_copy(x_hbm.at[pl.ds(base + ci * TCH, TCH)],
                             xbufs[p], isems[p])

        def in_wait(p):
            for ct in range(NCT):
                pltpu.make_async_copy(
                    lp_hbm.at[pl.ds(0, TCH), pl.ds(128 * ct, 128)],
                    bufs[p].at[ct], isems[p]).wait()
            pltpu.make_async_copy(x_hbm.at[pl.ds(0, TCH)],
                                  xbufs[p], isems[p]).wait()

        def out_start(ci, p):
            pltpu.async_copy(obufs[p],
                             out_hbm.at[pl.ds(base + ci * TCH, TCH)], osems[p])

        def out_wait(p):
            pltpu.make_async_copy(obufs[p],
                                  out_hbm.at[pl.ds(0, TCH)],
                                  osems[p]).wait()

        def process(p):
            buf, xbuf, obuf = bufs[p], xbufs[p], obufs[p]

            @plsc.parallel_loop(0, (TCH // L) * nch, unroll=1)
            def gbody(g):
                ch = g % nch
                tok0 = (g // nch) * L
                toks = tok0 + lane
                ct0 = 2 * ch
                acc_m = jnp.zeros((L,), jnp.float32)
                acc_s = jnp.zeros((L,), jnp.float32)
                for t in range(L):
                    vals = [buf[ct0 + j // 8, tok0 + t, pl.ds((j % 8) * L, L)]
                            for j in range(N_CLS // L)]
                    mx = jnp.max(_tree(jnp.maximum, vals))
                    ssum = jnp.sum(_tree(jnp.add,
                                         [jnp.exp(v - mx) for v in vals]))
                    sel = lane == t
                    acc_m = jnp.where(sel, mx, acc_m)
                    acc_s = jnp.where(sel, ssum, acc_s)
                lse = acc_m + _vlog(acc_s)
                chv = jnp.full((L,), 0, jnp.int32) + ch
                xi = plsc.load_gather(xbuf, [toks, chv])
                ctv = ct0 + lax.shift_right_logical(xi, 7)
                picked = plsc.load_gather(buf, [ctv, toks, xi & 127])
                plsc.store_scatter(obuf, [toks, chv], picked - lse)

        in_start(0, 0)
        in_start(1, 1)

        def cbody(k, carry):
            i = 2 * k
            for p in range(2):
                @pl.when(i >= 2)
                def _():
                    out_wait(p)

                in_wait(p)
                process(p)
                out_start(i + p, p)

                @pl.when(i + 2 + p < NCHK)
                def _():
                    in_start(i + 2 + p, p)
            return carry

        lax.fori_loop(0, NCHK // 2, cbody, 0)
        out_wait(0)
        out_wait(1)

    return sc_kernel


_BT = 1024


@functools.lru_cache(maxsize=None)
def _build_tc(ttc, off_blocks, nch):
    D = nch * N_CLS
    assert ttc % _BT == 0

    def body(x_ref, lp_ref, o_ref):
        blk = lp_ref[...]
        xb = x_ref[...]
        idx = lax.broadcasted_iota(jnp.int32, (_BT, N_CLS), 1)
        outs = []
        for ch in range(nch):
            sub = blk[:, ch * N_CLS:(ch + 1) * N_CLS]
            m = jnp.max(sub, axis=1, keepdims=True)
            e = jnp.exp(sub - m)
            lse = m + jnp.log(jnp.sum(e, axis=1, keepdims=True))
            sel = idx == xb[:, ch][:, None]
            picked = jnp.sum(jnp.where(sel, sub, 0.0), axis=1, keepdims=True)
            outs.append(picked - lse)
        o_ref[...] = jnp.concatenate(outs, axis=1)

    return pl.pallas_call(
        body,
        out_shape=jax.ShapeDtypeStruct((ttc, nch), jnp.float32),
        grid=(ttc // _BT,),
        in_specs=[
            pl.BlockSpec((_BT, nch), lambda i: (off_blocks + i, 0)),
            pl.BlockSpec((_BT, D), lambda i: (off_blocks + i, 0)),
        ],
        out_specs=pl.BlockSpec((_BT, nch), lambda i: (i, 0)),
    )


def kernel(x, dist_params):
    b, t, ch = x.shape
    tt = b * t
    lp = dist_params.reshape(tt, ch * N_CLS)
    xf = x.reshape(tt, ch)
    tsc = (tt * 18 // 32) // (NW * 64) * (NW * 64)
    sc_out = _build(tsc, ch)(lp, xf)
    tc_out = _build_tc(tt - tsc, tsc // _BT, ch)(xf, lp)
    return jnp.concatenate([sc_out, tc_out], axis=0).reshape(b, t, ch)
